# baseline (speedup 1.0000x reference)
import functools

import jax
import jax.numpy as jnp
from jax import lax
from jax.experimental import pallas as pl
from jax.experimental.pallas import tpu as pltpu
from jax.experimental.pallas import tpu_sc as plsc

N = 100000
B = 1024
K1 = 513
KP = 520
FEAT = 128
T_INV = 2.0
MOM = 0.5
EPS = 1e-07
CHUNK = 128
NCHUNK = 5
TAIL = KP - 4 * CHUNK
CPAD = CHUNK
NW = 32
BPW = B // NW
NBUF = 2
SPLIT = 56


def _rsqrt16(s):
    i = lax.bitcast_convert_type(s, jnp.int32)
    i = jnp.full((16,), 0x5F3759DF, jnp.int32) - lax.shift_right_logical(i, 1)
    x = lax.bitcast_convert_type(i, jnp.float32)
    h = s * 0.5
    for _ in range(4):
        x = x * (1.5 - h * x * x)
    return x


def _treesum(vs):
    while len(vs) > 1:
        vs = [a + b for a, b in zip(vs[::2], vs[1::2])]
    return vs[0]


def _lanesum16(x):
    iot = lax.iota(jnp.int32, 16)
    for k in (1, 2, 4, 8):
        x = x + x.at[iot ^ k].get(mode="promise_in_bounds")
    return x


def _normalize(ref):
    v = [ref[pl.ds(16 * j, 16)] for j in range(8)]
    tot = _lanesum16(_treesum([x * x for x in v]))
    sq = tot * _rsqrt16(tot)
    inv = 1.0 / (sq + EPS)
    return [x * inv for x in v]


def _upd_row(mrow, nvec, dst_ref):
    u = [MOM * a + (1.0 - MOM) * b for a, b in zip(mrow, nvec)]
    tot = _lanesum16(_treesum([x * x for x in u]))
    inv = _rsqrt16(tot)
    for j in range(8):
        dst_ref[pl.ds(16 * j, 16)] = u[j] * inv


def _dots(rows_ref, vec8, s_ref, c, tbuf, ngroups):
    colidx = lax.iota(jnp.int32, 16) * 17

    @pl.loop(0, ngroups)
    def _(g):
        for t in range(16):
            r = g * 16 + t
            prods = [rows_ref[r, pl.ds(16 * j, 16)] * vec8[j] for j in range(8)]
            plsc.store_scatter(tbuf, [colidx + t], _treesum(prods))
        cols = [tbuf[pl.ds(17 * i, 16)] for i in range(16)]
        s_ref[c, pl.ds(g * 16, 16)] = _treesum(cols)


def _sc_body(l_h, ab_h, y_h, idx_h, ml_h, mab_h,
             sA_h, sB_h, updl_h, updab_h, ysc_h,
             y_v, idx_v, lv, abv, rows0, rows1,
             sA_v, sB_v, updl_v, updab_v, yo_v, tbuf, g0, g1, gin):
    w = lax.axis_index("s") * 2 + lax.axis_index("c")
    base = w * BPW
    bufs = (rows0, rows1)
    sems = (g0, g1)
    ld = [pltpu.async_copy(y_h, y_v, gin),
          pltpu.async_copy(idx_h.at[pl.ds(base, BPW)], idx_v, gin),
          pltpu.async_copy(l_h.at[pl.ds(base, BPW)], lv, gin),
          pltpu.async_copy(ab_h.at[pl.ds(base, BPW)], abv, gin)]
    for cp in ld:
        cp.wait()

    def gather_desc(i, u):
        c, bank = divmod(u, 2)
        src = mab_h if bank == 0 else ml_h
        nrow = CHUNK if c < 4 else TAIL
        return pltpu.make_async_copy(
            src.at[idx_v.at[i, pl.ds(c * CHUNK, nrow)]],
            bufs[u % NBUF].at[pl.ds(0, nrow)], sems[u % NBUF])

    for u in range(NBUF):
        gather_desc(0, u).start()

    iot = lax.iota(jnp.int32, 16)
    for q in range(BPW // 16):
        yb = y_v[base // 16 + q]
        bvec = (base + q * 16) + iot

        def chk(t, acc):
            for p in range(16):
                vp = plsc.load_gather(
                    y_v, [jnp.full((16,), t, jnp.int32),
                          jnp.full((16,), p, jnp.int32)])
                hit = (yb == vp) & ((t * 16 + p) > bvec)
                acc = acc + jnp.where(hit, 1, 0)
            return acc

        acc = lax.fori_loop(0, B // 16, chk, jnp.zeros((16,), jnp.int32))
        yo_v[pl.ds(q * 16, 16)] = jnp.where(
            acc > 0, jnp.full((16,), N, jnp.int32), yb)


    @pl.loop(0, BPW)
    def _batch(i):
        ln = _normalize(lv.at[i])
        abn = _normalize(abv.at[i])

        saved = {}
        for u in range(2 * NCHUNK):
            c, bank = divmod(u, 2)
            buf = bufs[u % NBUF]
            gather_desc(i, u).wait()
            if u < 2:
                saved[bank] = [buf[0, pl.ds(16 * j, 16)] for j in range(8)]
            if u == 1:
                _upd_row(saved[1], ln, updl_v.at[i])
                _upd_row(saved[0], abn, updab_v.at[i])
            _dots(buf, ln if bank == 0 else abn,
                  sA_v.at[i] if bank == 0 else sB_v.at[i], c, tbuf,
                  8 if c < 4 else 1)
            nxt = u + NBUF
            if nxt < 2 * NCHUNK:
                gather_desc(i, nxt).start()
            else:
                @pl.when(i < BPW - 1)
                def _(nu=nxt - 2 * NCHUNK):
                    gather_desc(i + 1, nu).start()

    st = [pltpu.async_copy(sA_v, sA_h.at[pl.ds(base, BPW)], gin),
          pltpu.async_copy(sB_v, sB_h.at[pl.ds(base, BPW)], gin),
          pltpu.async_copy(updl_v, updl_h.at[pl.ds(base, BPW)], gin),
          pltpu.async_copy(updab_v, updab_h.at[pl.ds(base, BPW)], gin),
          pltpu.async_copy(yo_v, ysc_h.at[pl.ds(base, BPW)], gin)]
    for cp in st:
        cp.wait()


_SC_MESH = plsc.VectorSubcoreMesh(core_axis_name="c", subcore_axis_name="s")

_sc_call = pl.kernel(
    _sc_body,
    out_type=(
        jax.ShapeDtypeStruct((B, NCHUNK, CPAD), jnp.float32),
        jax.ShapeDtypeStruct((B, NCHUNK, CPAD), jnp.float32),
        jax.ShapeDtypeStruct((B, FEAT), jnp.float32),
        jax.ShapeDtypeStruct((B, FEAT), jnp.float32),
        jax.ShapeDtypeStruct((B,), jnp.int32),
    ),
    mesh=_SC_MESH,
    compiler_params=pltpu.CompilerParams(
        needs_layout_passes=False, use_tc_tiling_on_sc=False),
    scratch_types=[
        pltpu.VMEM((B // 16, 16), jnp.int32),
        pltpu.VMEM((BPW, KP), jnp.int32),
        pltpu.VMEM((BPW, FEAT), jnp.float32),
        pltpu.VMEM((BPW, FEAT), jnp.float32),
        pltpu.VMEM((CPAD, FEAT), jnp.float32),
        pltpu.VMEM((CPAD, FEAT), jnp.float32),
        pltpu.VMEM((BPW, NCHUNK, CPAD), jnp.float32),
        pltpu.VMEM((BPW, NCHUNK, CPAD), jnp.float32),
        pltpu.VMEM((BPW, FEAT), jnp.float32),
        pltpu.VMEM((BPW, FEAT), jnp.float32),
        pltpu.VMEM((BPW,), jnp.int32),
        pltpu.VMEM((16 * 17, ), jnp.float32),
        pltpu.SemaphoreType.DMA,
        pltpu.SemaphoreType.DMA,
        pltpu.SemaphoreType.DMA,
    ],
)


def _unpack_scores(ref):
    s = ref[...]
    s513 = jnp.concatenate(
        [s[:, :4, :].reshape(B, 4 * CHUNK), s[:, 4, :1]], axis=1)
    return jnp.exp(s513 * T_INV)


CROWS = 12500


def _tc_body(sA_ref, sB_ref, updl_ref, updab_ref, ysc_ref, ml_any, mab_any,
             outl_ref, outab_ref, newl_any, newab_any, sem0, sem1, semc):
    def copy_desc(srcb, dstb, cc):
        return pltpu.make_async_copy(
            srcb.at[pl.ds(cc * CROWS, CROWS)],
            dstb.at[pl.ds(cc * CROWS, CROWS)], semc)

    pairs = ((ml_any, newl_any), (mab_any, newab_any))
    for srcb, dstb in pairs:
        for cc in range(N // CROWS):
            copy_desc(srcb, dstb, cc).start()

    def put(b, _):
        yb = ysc_ref[b]

        @pl.when(yb < N)
        def _():
            pltpu.make_async_copy(updl_ref.at[b], newl_any.at[yb], sem0).start()
            pltpu.make_async_copy(updab_ref.at[b], newab_any.at[yb], sem1).start()
        return 0

    pA = _unpack_scores(sA_ref)
    outl_ref[...] = pA / (jnp.sum(pA) * (float(N) / (B * K1)))
    pB = _unpack_scores(sB_ref)
    outab_ref[...] = pB / (jnp.sum(pB) * (float(N) / (B * K1)))

    for srcb, dstb in pairs:
        for cc in range(N // CROWS):
            copy_desc(srcb, dstb, cc).wait()

    lax.fori_loop(0, B, put, 0)

    def drain(b, _):
        yb = ysc_ref[b]

        @pl.when(yb < N)
        def _():
            pltpu.make_async_copy(updl_ref.at[b], newl_any.at[yb], sem0).wait()
            pltpu.make_async_copy(updab_ref.at[b], newab_any.at[yb], sem1).wait()
        return 0

    lax.fori_loop(0, B, drain, 0)


_tc_call = pl.pallas_call(
    _tc_body,
    out_shape=[
        jax.ShapeDtypeStruct((B, K1), jnp.float32),
        jax.ShapeDtypeStruct((B, K1), jnp.float32),
        jax.ShapeDtypeStruct((N, FEAT), jnp.float32),
        jax.ShapeDtypeStruct((N, FEAT), jnp.float32),
    ],
    in_specs=[
        pl.BlockSpec(memory_space=pltpu.MemorySpace.VMEM),
        pl.BlockSpec(memory_space=pltpu.MemorySpace.VMEM),
        pl.BlockSpec(memory_space=pltpu.MemorySpace.VMEM),
        pl.BlockSpec(memory_space=pltpu.MemorySpace.VMEM),
        pl.BlockSpec(memory_space=pltpu.MemorySpace.SMEM),
        pl.BlockSpec(memory_space=pltpu.MemorySpace.HBM),
        pl.BlockSpec(memory_space=pltpu.MemorySpace.HBM),
    ],
    out_specs=[
        pl.BlockSpec(memory_space=pltpu.MemorySpace.VMEM),
        pl.BlockSpec(memory_space=pltpu.MemorySpace.VMEM),
        pl.BlockSpec(memory_space=pltpu.MemorySpace.HBM),
        pl.BlockSpec(memory_space=pltpu.MemorySpace.HBM),
    ],
    scratch_shapes=[pltpu.SemaphoreType.DMA, pltpu.SemaphoreType.DMA,
                    pltpu.SemaphoreType.DMA],
)


def kernel(l, ab, y, idx, memory_l, memory_ab):
    y = y.astype(jnp.int32)
    idx = idx.astype(jnp.int32)
    idx_p = jnp.concatenate(
        [idx, jnp.zeros((B, KP - K1), jnp.int32)], axis=1)
    sA, sB, updl, updab, ysc = _sc_call(
        l, ab, y.reshape(B // 16, 16), idx_p, memory_l, memory_ab)
    out_l, out_ab, new_l, new_ab = _tc_call(
        sA, sB, updl, updab, ysc, memory_l, memory_ab)
    return (out_l[..., None], out_ab[..., None], new_l, new_ab)


# speedup vs baseline: 5.4347x; 5.4347x over previous
import functools

import jax
import jax.numpy as jnp
from jax import lax
from jax.experimental import pallas as pl
from jax.experimental.pallas import tpu as pltpu
from jax.experimental.pallas import tpu_sc as plsc

N = 100000
B = 1024
K1 = 513
KP = 520
FEAT = 128
T_INV = 2.0
MOM = 0.5
EPS = 1e-07
CHUNK = 128
NCHUNK = 5
TAIL = KP - 4 * CHUNK
CPAD = CHUNK
NW = 32
BPW = B // NW
NBUF = 2
SPLIT = 56


def _rsqrt16(s):
    i = lax.bitcast_convert_type(s, jnp.int32)
    i = jnp.full((16,), 0x5F3759DF, jnp.int32) - lax.shift_right_logical(i, 1)
    x = lax.bitcast_convert_type(i, jnp.float32)
    h = s * 0.5
    for _ in range(4):
        x = x * (1.5 - h * x * x)
    return x


def _treesum(vs):
    while len(vs) > 1:
        vs = [a + b for a, b in zip(vs[::2], vs[1::2])]
    return vs[0]


def _lanesum16(x):
    iot = lax.iota(jnp.int32, 16)
    for k in (1, 2, 4, 8):
        x = x + x.at[iot ^ k].get(mode="promise_in_bounds")
    return x


def _normalize(ref):
    v = [ref[pl.ds(16 * j, 16)] for j in range(8)]
    tot = _lanesum16(_treesum([x * x for x in v]))
    sq = tot * _rsqrt16(tot)
    inv = 1.0 / (sq + EPS)
    return [x * inv for x in v]


def _upd_row(mrow, nvec, dst_ref):
    u = [MOM * a + (1.0 - MOM) * b for a, b in zip(mrow, nvec)]
    tot = _lanesum16(_treesum([x * x for x in u]))
    inv = _rsqrt16(tot)
    for j in range(8):
        dst_ref[pl.ds(16 * j, 16)] = u[j] * inv


def _dots(rows_ref, vec8, s_ref, c, tbuf, ngroups):
    colidx = lax.iota(jnp.int32, 16) * 17

    @pl.loop(0, ngroups)
    def _(g):
        for t in range(16):
            r = g * 16 + t
            prods = [rows_ref[r, pl.ds(16 * j, 16)] * vec8[j] for j in range(8)]
            plsc.store_scatter(tbuf, [colidx + t], _treesum(prods))
        cols = [tbuf[pl.ds(17 * i, 16)] for i in range(16)]
        s_ref[c, pl.ds(g * 16, 16)] = _treesum(cols)


def _sc_body(l_h, ab_h, y_h, idx_h, ml_h, mab_h,
             sA_h, sB_h, updl_h, updab_h, ysc_h,
             y_v, idx_v, lv, abv, rows0, rows1,
             sA_v, sB_v, updl_v, updab_v, yo_v, tbuf, g0, g1, gin):
    w = lax.axis_index("s") * 2 + lax.axis_index("c")
    base = w * BPW
    bufs = (rows0, rows1)
    sems = (g0, g1)
    ld = [pltpu.async_copy(y_h, y_v, gin),
          pltpu.async_copy(idx_h.at[pl.ds(base, BPW)], idx_v, gin),
          pltpu.async_copy(l_h.at[pl.ds(base, BPW)], lv, gin),
          pltpu.async_copy(ab_h.at[pl.ds(base, BPW)], abv, gin)]
    for cp in ld:
        cp.wait()

    def gather_desc(i, u):
        c, bank = divmod(u, 2)
        src = mab_h if bank == 0 else ml_h
        nrow = CHUNK if c < 4 else TAIL
        return pltpu.make_async_copy(
            src.at[idx_v.at[i, pl.ds(c * CHUNK, nrow)]],
            bufs[u % NBUF].at[pl.ds(0, nrow)], sems[u % NBUF])

    for u in range(NBUF):
        gather_desc(0, u).start()

    iot = lax.iota(jnp.int32, 16)
    for q in range(BPW // 16):
        yb = y_v[base // 16 + q]
        bvec = (base + q * 16) + iot

        def chk(t, acc):
            for p in range(16):
                vp = plsc.load_gather(
                    y_v, [jnp.full((16,), t, jnp.int32),
                          jnp.full((16,), p, jnp.int32)])
                hit = (yb == vp) & ((t * 16 + p) > bvec)
                acc = acc + jnp.where(hit, 1, 0)
            return acc

        acc = lax.fori_loop(0, B // 16, chk, jnp.zeros((16,), jnp.int32))
        yo_v[pl.ds(q * 16, 16)] = jnp.where(
            acc > 0, jnp.full((16,), N, jnp.int32), yb)


    @pl.loop(0, BPW)
    def _batch(i):
        ln = _normalize(lv.at[i])
        abn = _normalize(abv.at[i])

        saved = {}
        for u in range(2 * NCHUNK):
            c, bank = divmod(u, 2)
            buf = bufs[u % NBUF]
            gather_desc(i, u).wait()
            if u < 2:
                saved[bank] = [buf[0, pl.ds(16 * j, 16)] for j in range(8)]
            if u == 1:
                _upd_row(saved[1], ln, updl_v.at[i])
                _upd_row(saved[0], abn, updab_v.at[i])
            _dots(buf, ln if bank == 0 else abn,
                  sA_v.at[i] if bank == 0 else sB_v.at[i], c, tbuf,
                  8 if c < 4 else 1)
            nxt = u + NBUF
            if nxt < 2 * NCHUNK:
                gather_desc(i, nxt).start()
            else:
                @pl.when(i < BPW - 1)
                def _(nu=nxt - 2 * NCHUNK):
                    gather_desc(i + 1, nu).start()

    st = [pltpu.async_copy(sA_v, sA_h.at[pl.ds(base, BPW)], gin),
          pltpu.async_copy(sB_v, sB_h.at[pl.ds(base, BPW)], gin),
          pltpu.async_copy(updl_v, updl_h.at[pl.ds(base, BPW)], gin),
          pltpu.async_copy(updab_v, updab_h.at[pl.ds(base, BPW)], gin),
          pltpu.async_copy(yo_v, ysc_h.at[pl.ds(base, BPW)], gin)]
    for cp in st:
        cp.wait()


_SC_MESH = plsc.VectorSubcoreMesh(core_axis_name="c", subcore_axis_name="s")

_sc_call = pl.kernel(
    _sc_body,
    out_type=(
        jax.ShapeDtypeStruct((B, NCHUNK, CPAD), jnp.float32),
        jax.ShapeDtypeStruct((B, NCHUNK, CPAD), jnp.float32),
        jax.ShapeDtypeStruct((B, FEAT), jnp.float32),
        jax.ShapeDtypeStruct((B, FEAT), jnp.float32),
        jax.ShapeDtypeStruct((B,), jnp.int32),
    ),
    mesh=_SC_MESH,
    compiler_params=pltpu.CompilerParams(
        needs_layout_passes=False, use_tc_tiling_on_sc=False),
    scratch_types=[
        pltpu.VMEM((B // 16, 16), jnp.int32),
        pltpu.VMEM((BPW, KP), jnp.int32),
        pltpu.VMEM((BPW, FEAT), jnp.float32),
        pltpu.VMEM((BPW, FEAT), jnp.float32),
        pltpu.VMEM((CPAD, FEAT), jnp.float32),
        pltpu.VMEM((CPAD, FEAT), jnp.float32),
        pltpu.VMEM((BPW, NCHUNK, CPAD), jnp.float32),
        pltpu.VMEM((BPW, NCHUNK, CPAD), jnp.float32),
        pltpu.VMEM((BPW, FEAT), jnp.float32),
        pltpu.VMEM((BPW, FEAT), jnp.float32),
        pltpu.VMEM((BPW,), jnp.int32),
        pltpu.VMEM((16 * 17, ), jnp.float32),
        pltpu.SemaphoreType.DMA,
        pltpu.SemaphoreType.DMA,
        pltpu.SemaphoreType.DMA,
    ],
)


def _unpack_scores(ref):
    s = ref[...]
    s513 = jnp.concatenate(
        [s[:, :4, :].reshape(B, 4 * CHUNK), s[:, 4, :1]], axis=1)
    return jnp.exp(s513 * T_INV)


def _tc_body(sA_ref, sB_ref, updl_ref, updab_ref, ysc_ref, ml_any, mab_any,
             outl_ref, outab_ref, newl_any, newab_any, sem0, sem1):
    def put(b, _):
        yb = ysc_ref[b]

        @pl.when(yb < N)
        def _():
            pltpu.make_async_copy(updl_ref.at[b], newl_any.at[yb], sem0).start()
            pltpu.make_async_copy(updab_ref.at[b], newab_any.at[yb], sem1).start()
        return 0

    lax.fori_loop(0, B, put, 0)

    pA = _unpack_scores(sA_ref)
    outl_ref[...] = pA / (jnp.sum(pA) * (float(N) / (B * K1)))
    pB = _unpack_scores(sB_ref)
    outab_ref[...] = pB / (jnp.sum(pB) * (float(N) / (B * K1)))

    def drain(b, _):
        yb = ysc_ref[b]

        @pl.when(yb < N)
        def _():
            pltpu.make_async_copy(updl_ref.at[b], newl_any.at[yb], sem0).wait()
            pltpu.make_async_copy(updab_ref.at[b], newab_any.at[yb], sem1).wait()
        return 0

    lax.fori_loop(0, B, drain, 0)


_tc_call = pl.pallas_call(
    _tc_body,
    out_shape=[
        jax.ShapeDtypeStruct((B, K1), jnp.float32),
        jax.ShapeDtypeStruct((B, K1), jnp.float32),
        jax.ShapeDtypeStruct((N, FEAT), jnp.float32),
        jax.ShapeDtypeStruct((N, FEAT), jnp.float32),
    ],
    in_specs=[
        pl.BlockSpec(memory_space=pltpu.MemorySpace.VMEM),
        pl.BlockSpec(memory_space=pltpu.MemorySpace.VMEM),
        pl.BlockSpec(memory_space=pltpu.MemorySpace.VMEM),
        pl.BlockSpec(memory_space=pltpu.MemorySpace.VMEM),
        pl.BlockSpec(memory_space=pltpu.MemorySpace.SMEM),
        pl.BlockSpec(memory_space=pltpu.MemorySpace.HBM),
        pl.BlockSpec(memory_space=pltpu.MemorySpace.HBM),
    ],
    out_specs=[
        pl.BlockSpec(memory_space=pltpu.MemorySpace.VMEM),
        pl.BlockSpec(memory_space=pltpu.MemorySpace.VMEM),
        pl.BlockSpec(memory_space=pltpu.MemorySpace.HBM),
        pl.BlockSpec(memory_space=pltpu.MemorySpace.HBM),
    ],
    scratch_shapes=[pltpu.SemaphoreType.DMA, pltpu.SemaphoreType.DMA],
    input_output_aliases={5: 2, 6: 3},
)


def kernel(l, ab, y, idx, memory_l, memory_ab):
    y = y.astype(jnp.int32)
    idx = idx.astype(jnp.int32)
    idx_p = jnp.concatenate(
        [idx, jnp.zeros((B, KP - K1), jnp.int32)], axis=1)
    sA, sB, updl, updab, ysc = _sc_call(
        l, ab, y.reshape(B // 16, 16), idx_p, memory_l, memory_ab)
    out_l, out_ab, new_l, new_ab = _tc_call(
        sA, sB, updl, updab, ysc, memory_l, memory_ab)
    return (out_l[..., None], out_ab[..., None], new_l, new_ab)


# speedup vs baseline: 5.4435x; 1.0016x over previous
import jax
import jax.numpy as jnp
from jax import lax
from jax.experimental import pallas as pl
from jax.experimental.pallas import tpu as pltpu
from jax.experimental.pallas import tpu_sc as plsc

N = 100000
B = 1024
K1 = 513
KP = 520
FEAT = 128
T_INV = 2.0
MOM = 0.5
EPS = 1e-07
CHUNK = 128
NCHUNK = 5
TAIL = KP - 4 * CHUNK
CPAD = CHUNK
NW = 32
BPW = B // NW
NBUF = 2


def _rsqrt16(s):
    i = lax.bitcast_convert_type(s, jnp.int32)
    i = jnp.full((16,), 0x5F3759DF, jnp.int32) - lax.shift_right_logical(i, 1)
    x = lax.bitcast_convert_type(i, jnp.float32)
    h = s * 0.5
    for _ in range(4):
        x = x * (1.5 - h * x * x)
    return x


def _treesum(vs):
    while len(vs) > 1:
        vs = [a + b for a, b in zip(vs[::2], vs[1::2])]
    return vs[0]


def _lanesum16(x):
    iot = lax.iota(jnp.int32, 16)
    for k in (1, 2, 4, 8):
        x = x + x.at[iot ^ k].get(mode="promise_in_bounds")
    return x


def _normalize(ref):
    v = [ref[pl.ds(16 * j, 16)] for j in range(8)]
    tot = _lanesum16(_treesum([x * x for x in v]))
    sq = tot * _rsqrt16(tot)
    inv = 1.0 / (sq + EPS)
    return [x * inv for x in v]


def _upd_row(mrow, nvec, dst_ref):
    u = [MOM * a + (1.0 - MOM) * b for a, b in zip(mrow, nvec)]
    tot = _lanesum16(_treesum([x * x for x in u]))
    inv = _rsqrt16(tot)
    for j in range(8):
        dst_ref[pl.ds(16 * j, 16)] = u[j] * inv


def _dots(rows_ref, vec8, s_ref, c, tbuf, ngroups):
    colidx = lax.iota(jnp.int32, 16) * 17

    @pl.loop(0, ngroups)
    def _(g):
        for t in range(16):
            r = g * 16 + t
            prods = [rows_ref[r, pl.ds(16 * j, 16)] * vec8[j] for j in range(8)]
            plsc.store_scatter(tbuf, [colidx + t], _treesum(prods))
        cols = [tbuf[pl.ds(17 * i, 16)] for i in range(16)]
        s_ref[c, pl.ds(g * 16, 16)] = _treesum(cols)


def _sc_body(l_h, ab_h, y_h, idx_h, ml_h, mab_h,
             sA_h, sB_h, updl_h, updab_h, ysc_h,
             y_v, idx_v, lv, abv, rows0, rows1,
             sA_v, sB_v, updl_v, updab_v, yo_v, tbuf, g0, g1, gin):
    w = lax.axis_index("s") * 2 + lax.axis_index("c")
    base = w * BPW
    bufs = (rows0, rows1)
    sems = (g0, g1)
    ld = [pltpu.async_copy(y_h, y_v, gin),
          pltpu.async_copy(idx_h.at[pl.ds(base, BPW)], idx_v, gin),
          pltpu.async_copy(l_h.at[pl.ds(base, BPW)], lv, gin),
          pltpu.async_copy(ab_h.at[pl.ds(base, BPW)], abv, gin)]
    for cp in ld:
        cp.wait()

    def gather_desc(i, u):
        c, bank = divmod(u, 2)
        src = mab_h if bank == 0 else ml_h
        nrow = CHUNK if c < 4 else TAIL
        return pltpu.make_async_copy(
            src.at[idx_v.at[i, pl.ds(c * CHUNK, nrow)]],
            bufs[u % NBUF].at[pl.ds(0, nrow)], sems[u % NBUF])

    for u in range(NBUF):
        gather_desc(0, u).start()

    iot = lax.iota(jnp.int32, 16)
    for q in range(BPW // 16):
        yb = y_v[base // 16 + q]
        bvec = (base + q * 16) + iot

        def chk(t, acc):
            for p in range(16):
                vp = plsc.load_gather(
                    y_v, [jnp.full((16,), t, jnp.int32),
                          jnp.full((16,), p, jnp.int32)])
                hit = (yb == vp) & ((t * 16 + p) > bvec)
                acc = acc + jnp.where(hit, 1, 0)
            return acc

        acc = lax.fori_loop(0, B // 16, chk, jnp.zeros((16,), jnp.int32))
        yo_v[pl.ds(q * 16, 16)] = jnp.where(
            acc > 0, jnp.full((16,), N, jnp.int32), yb)


    @pl.loop(0, BPW)
    def _batch(i):
        ln = _normalize(lv.at[i])
        abn = _normalize(abv.at[i])

        saved = {}
        for u in range(2 * NCHUNK):
            c, bank = divmod(u, 2)
            buf = bufs[u % NBUF]
            gather_desc(i, u).wait()
            if u < 2:
                saved[bank] = [buf[0, pl.ds(16 * j, 16)] for j in range(8)]
            if u == 1:
                _upd_row(saved[1], ln, updl_v.at[i])
                _upd_row(saved[0], abn, updab_v.at[i])
            _dots(buf, ln if bank == 0 else abn,
                  sA_v.at[i] if bank == 0 else sB_v.at[i], c, tbuf,
                  8 if c < 4 else 1)
            nxt = u + NBUF
            if nxt < 2 * NCHUNK:
                gather_desc(i, nxt).start()
            else:
                @pl.when(i < BPW - 1)
                def _(nu=nxt - 2 * NCHUNK):
                    gather_desc(i + 1, nu).start()

    st = [pltpu.async_copy(sA_v, sA_h.at[pl.ds(base, BPW)], gin),
          pltpu.async_copy(sB_v, sB_h.at[pl.ds(base, BPW)], gin),
          pltpu.async_copy(updl_v, updl_h.at[pl.ds(base, BPW)], gin),
          pltpu.async_copy(updab_v, updab_h.at[pl.ds(base, BPW)], gin),
          pltpu.async_copy(yo_v, ysc_h.at[pl.ds(base, BPW)], gin)]
    for cp in st:
        cp.wait()


_SC_MESH = plsc.VectorSubcoreMesh(core_axis_name="c", subcore_axis_name="s")

_sc_call = pl.kernel(
    _sc_body,
    out_type=(
        jax.ShapeDtypeStruct((B, NCHUNK, CPAD), jnp.float32),
        jax.ShapeDtypeStruct((B, NCHUNK, CPAD), jnp.float32),
        jax.ShapeDtypeStruct((B, FEAT), jnp.float32),
        jax.ShapeDtypeStruct((B, FEAT), jnp.float32),
        jax.ShapeDtypeStruct((B,), jnp.int32),
    ),
    mesh=_SC_MESH,
    compiler_params=pltpu.CompilerParams(
        needs_layout_passes=False, use_tc_tiling_on_sc=False),
    scratch_types=[
        pltpu.VMEM((B // 16, 16), jnp.int32),
        pltpu.VMEM((BPW, KP), jnp.int32),
        pltpu.VMEM((BPW, FEAT), jnp.float32),
        pltpu.VMEM((BPW, FEAT), jnp.float32),
        pltpu.VMEM((CPAD, FEAT), jnp.float32),
        pltpu.VMEM((CPAD, FEAT), jnp.float32),
        pltpu.VMEM((BPW, NCHUNK, CPAD), jnp.float32),
        pltpu.VMEM((BPW, NCHUNK, CPAD), jnp.float32),
        pltpu.VMEM((BPW, FEAT), jnp.float32),
        pltpu.VMEM((BPW, FEAT), jnp.float32),
        pltpu.VMEM((BPW,), jnp.int32),
        pltpu.VMEM((16 * 17, ), jnp.float32),
        pltpu.SemaphoreType.DMA,
        pltpu.SemaphoreType.DMA,
        pltpu.SemaphoreType.DMA,
    ],
)


def _unpack_scores(ref):
    s = ref[...]
    s513 = jnp.concatenate(
        [s[:, :4, :].reshape(B, 4 * CHUNK), s[:, 4, :1]], axis=1)
    return jnp.exp(s513 * T_INV)


def _tc_body(sA_ref, sB_ref, updl_ref, updab_ref, ysc_ref, ml_any, mab_any,
             outl_ref, outab_ref, newl_any, newab_any, sem0, sem1):
    def put(b, _):
        yb = ysc_ref[b]

        @pl.when(yb < N)
        def _():
            pltpu.make_async_copy(updl_ref.at[b], newl_any.at[yb], sem0).start()
            pltpu.make_async_copy(updab_ref.at[b], newab_any.at[yb], sem1).start()
        return 0

    lax.fori_loop(0, B, put, 0)

    pA = _unpack_scores(sA_ref)
    outl_ref[...] = pA / (jnp.sum(pA) * (float(N) / (B * K1)))
    pB = _unpack_scores(sB_ref)
    outab_ref[...] = pB / (jnp.sum(pB) * (float(N) / (B * K1)))

    def drain(b, _):
        yb = ysc_ref[b]

        @pl.when(yb < N)
        def _():
            pltpu.make_async_copy(updl_ref.at[b], newl_any.at[yb], sem0).wait()
            pltpu.make_async_copy(updab_ref.at[b], newab_any.at[yb], sem1).wait()
        return 0

    lax.fori_loop(0, B, drain, 0)


_tc_call = pl.pallas_call(
    _tc_body,
    out_shape=[
        jax.ShapeDtypeStruct((B, K1), jnp.float32),
        jax.ShapeDtypeStruct((B, K1), jnp.float32),
        jax.ShapeDtypeStruct((N, FEAT), jnp.float32),
        jax.ShapeDtypeStruct((N, FEAT), jnp.float32),
    ],
    in_specs=[
        pl.BlockSpec(memory_space=pltpu.MemorySpace.VMEM),
        pl.BlockSpec(memory_space=pltpu.MemorySpace.VMEM),
        pl.BlockSpec(memory_space=pltpu.MemorySpace.VMEM),
        pl.BlockSpec(memory_space=pltpu.MemorySpace.VMEM),
        pl.BlockSpec(memory_space=pltpu.MemorySpace.SMEM),
        pl.BlockSpec(memory_space=pltpu.MemorySpace.HBM),
        pl.BlockSpec(memory_space=pltpu.MemorySpace.HBM),
    ],
    out_specs=[
        pl.BlockSpec(memory_space=pltpu.MemorySpace.VMEM),
        pl.BlockSpec(memory_space=pltpu.MemorySpace.VMEM),
        pl.BlockSpec(memory_space=pltpu.MemorySpace.HBM),
        pl.BlockSpec(memory_space=pltpu.MemorySpace.HBM),
    ],
    scratch_shapes=[pltpu.SemaphoreType.DMA, pltpu.SemaphoreType.DMA],
    input_output_aliases={5: 2, 6: 3},
)


def kernel(l, ab, y, idx, memory_l, memory_ab):
    y = y.astype(jnp.int32)
    idx = idx.astype(jnp.int32)
    idx_p = jnp.concatenate(
        [idx, jnp.zeros((B, KP - K1), jnp.int32)], axis=1)
    sA, sB, updl, updab, ysc = _sc_call(
        l, ab, y.reshape(B // 16, 16), idx_p, memory_l, memory_ab)
    out_l, out_ab, new_l, new_ab = _tc_call(
        sA, sB, updl, updab, ysc, memory_l, memory_ab)
    return (out_l[..., None], out_ab[..., None], new_l, new_ab)


# speedup vs baseline: 5.7463x; 1.0556x over previous
import jax
import jax.numpy as jnp
from jax import lax
from jax.experimental import pallas as pl
from jax.experimental.pallas import tpu as pltpu
from jax.experimental.pallas import tpu_sc as plsc

N = 100000
B = 1024
K1 = 513
KP = 520
FEAT = 128
T_INV = 2.0
MOM = 0.5
EPS = 1e-07
CHUNK = 128
NCHUNK = 5
TAIL = 1
CPAD = CHUNK
NW = 32
BPW = B // NW
NBUF = 2


def _rsqrt16(s):
    i = lax.bitcast_convert_type(s, jnp.int32)
    i = jnp.full((16,), 0x5F3759DF, jnp.int32) - lax.shift_right_logical(i, 1)
    x = lax.bitcast_convert_type(i, jnp.float32)
    h = s * 0.5
    for _ in range(4):
        x = x * (1.5 - h * x * x)
    return x


def _treesum(vs):
    while len(vs) > 1:
        vs = [a + b for a, b in zip(vs[::2], vs[1::2])]
    return vs[0]


def _lanesum16(x):
    iot = lax.iota(jnp.int32, 16)
    for k in (1, 2, 4, 8):
        x = x + x.at[iot ^ k].get(mode="promise_in_bounds")
    return x


def _normalize(ref):
    v = [ref[pl.ds(16 * j, 16)] for j in range(8)]
    tot = _lanesum16(_treesum([x * x for x in v]))
    sq = tot * _rsqrt16(tot)
    inv = 1.0 / (sq + EPS)
    return [x * inv for x in v]


def _upd_row(mrow, nvec, dst_ref):
    u = [MOM * a + (1.0 - MOM) * b for a, b in zip(mrow, nvec)]
    tot = _lanesum16(_treesum([x * x for x in u]))
    inv = _rsqrt16(tot)
    for j in range(8):
        dst_ref[pl.ds(16 * j, 16)] = u[j] * inv


def _dots(rows_ref, vec8, s_ref, c, tbuf, ngroups):
    colidx = lax.iota(jnp.int32, 16) * 17

    @pl.loop(0, ngroups)
    def _(g):
        for t in range(16):
            r = g * 16 + t
            prods = [rows_ref[r, pl.ds(16 * j, 16)] * vec8[j] for j in range(8)]
            plsc.store_scatter(tbuf, [colidx + t], _treesum(prods))
        cols = [tbuf[pl.ds(17 * i, 16)] for i in range(16)]
        s_ref[c, pl.ds(g * 16, 16)] = _treesum(cols)


def _sc_body(l_h, ab_h, y_h, idx_h, ml_h, mab_h,
             sA_h, sB_h, updl_h, updab_h, ysc_h,
             y_v, idx_v, lv, abv, rows0, rows1,
             sA_v, sB_v, updl_v, updab_v, yo_v, tbuf, g0, g1, gin):
    w = lax.axis_index("s") * 2 + lax.axis_index("c")
    base = w * BPW
    bufs = (rows0, rows1)
    sems = (g0, g1)
    ld = [pltpu.async_copy(y_h, y_v, gin),
          pltpu.async_copy(idx_h.at[pl.ds(base, BPW)], idx_v, gin),
          pltpu.async_copy(l_h.at[pl.ds(base, BPW)], lv, gin),
          pltpu.async_copy(ab_h.at[pl.ds(base, BPW)], abv, gin)]
    for cp in ld:
        cp.wait()

    def gather_desc(i, u):
        c, bank = divmod(u, 2)
        src = mab_h if bank == 0 else ml_h
        nrow = CHUNK if c < 4 else TAIL
        return pltpu.make_async_copy(
            src.at[idx_v.at[i, pl.ds(c * CHUNK, nrow)]],
            bufs[u % NBUF].at[pl.ds(0, nrow)], sems[u % NBUF])

    for u in range(NBUF):
        gather_desc(0, u).start()

    iot = lax.iota(jnp.int32, 16)
    for q in range(BPW // 16):
        yb = y_v[base // 16 + q]
        bvec = (base + q * 16) + iot

        def chk(t, acc):
            for p in range(16):
                vp = plsc.load_gather(
                    y_v, [jnp.full((16,), t, jnp.int32),
                          jnp.full((16,), p, jnp.int32)])
                hit = (yb == vp) & ((t * 16 + p) > bvec)
                acc = acc + jnp.where(hit, 1, 0)
            return acc

        acc = lax.fori_loop(0, B // 16, chk, jnp.zeros((16,), jnp.int32))
        yo_v[pl.ds(q * 16, 16)] = jnp.where(
            acc > 0, jnp.full((16,), N, jnp.int32), yb)


    @pl.loop(0, BPW)
    def _batch(i):
        ln = _normalize(lv.at[i])
        abn = _normalize(abv.at[i])

        saved = {}
        for u in range(2 * NCHUNK):
            c, bank = divmod(u, 2)
            buf = bufs[u % NBUF]
            gather_desc(i, u).wait()
            if u < 2:
                saved[bank] = [buf[0, pl.ds(16 * j, 16)] for j in range(8)]
            if u == 1:
                _upd_row(saved[1], ln, updl_v.at[i])
                _upd_row(saved[0], abn, updab_v.at[i])
            _dots(buf, ln if bank == 0 else abn,
                  sA_v.at[i] if bank == 0 else sB_v.at[i], c, tbuf,
                  8 if c < 4 else 1)
            nxt = u + NBUF
            if nxt < 2 * NCHUNK:
                gather_desc(i, nxt).start()
            else:
                @pl.when(i < BPW - 1)
                def _(nu=nxt - 2 * NCHUNK):
                    gather_desc(i + 1, nu).start()

    st = [pltpu.async_copy(sA_v, sA_h.at[pl.ds(base, BPW)], gin),
          pltpu.async_copy(sB_v, sB_h.at[pl.ds(base, BPW)], gin),
          pltpu.async_copy(updl_v, updl_h.at[pl.ds(base, BPW)], gin),
          pltpu.async_copy(updab_v, updab_h.at[pl.ds(base, BPW)], gin),
          pltpu.async_copy(yo_v, ysc_h.at[pl.ds(base, BPW)], gin)]
    for cp in st:
        cp.wait()


_SC_MESH = plsc.VectorSubcoreMesh(core_axis_name="c", subcore_axis_name="s")

_sc_call = pl.kernel(
    _sc_body,
    out_type=(
        jax.ShapeDtypeStruct((B, NCHUNK, CPAD), jnp.float32),
        jax.ShapeDtypeStruct((B, NCHUNK, CPAD), jnp.float32),
        jax.ShapeDtypeStruct((B, FEAT), jnp.float32),
        jax.ShapeDtypeStruct((B, FEAT), jnp.float32),
        jax.ShapeDtypeStruct((B,), jnp.int32),
    ),
    mesh=_SC_MESH,
    compiler_params=pltpu.CompilerParams(
        needs_layout_passes=False, use_tc_tiling_on_sc=False),
    scratch_types=[
        pltpu.VMEM((B // 16, 16), jnp.int32),
        pltpu.VMEM((BPW, KP), jnp.int32),
        pltpu.VMEM((BPW, FEAT), jnp.float32),
        pltpu.VMEM((BPW, FEAT), jnp.float32),
        pltpu.VMEM((CPAD, FEAT), jnp.float32),
        pltpu.VMEM((CPAD, FEAT), jnp.float32),
        pltpu.VMEM((BPW, NCHUNK, CPAD), jnp.float32),
        pltpu.VMEM((BPW, NCHUNK, CPAD), jnp.float32),
        pltpu.VMEM((BPW, FEAT), jnp.float32),
        pltpu.VMEM((BPW, FEAT), jnp.float32),
        pltpu.VMEM((BPW,), jnp.int32),
        pltpu.VMEM((16 * 17, ), jnp.float32),
        pltpu.SemaphoreType.DMA,
        pltpu.SemaphoreType.DMA,
        pltpu.SemaphoreType.DMA,
    ],
)


def _unpack_scores(ref):
    s = ref[...]
    s513 = jnp.concatenate(
        [s[:, :4, :].reshape(B, 4 * CHUNK), s[:, 4, :1]], axis=1)
    return jnp.exp(s513 * T_INV)


def _tc_body(sA_ref, sB_ref, updl_ref, updab_ref, ysc_ref, ml_any, mab_any,
             outl_ref, outab_ref, newl_any, newab_any, sem0, sem1):
    def put(b, _):
        yb = ysc_ref[b]

        @pl.when(yb < N)
        def _():
            pltpu.make_async_copy(updl_ref.at[b], newl_any.at[yb], sem0).start()
            pltpu.make_async_copy(updab_ref.at[b], newab_any.at[yb], sem1).start()
        return 0

    lax.fori_loop(0, B, put, 0)

    pA = _unpack_scores(sA_ref)
    outl_ref[...] = pA / (jnp.sum(pA) * (float(N) / (B * K1)))
    pB = _unpack_scores(sB_ref)
    outab_ref[...] = pB / (jnp.sum(pB) * (float(N) / (B * K1)))

    def drain(b, _):
        yb = ysc_ref[b]

        @pl.when(yb < N)
        def _():
            pltpu.make_async_copy(updl_ref.at[b], newl_any.at[yb], sem0).wait()
            pltpu.make_async_copy(updab_ref.at[b], newab_any.at[yb], sem1).wait()
        return 0

    lax.fori_loop(0, B, drain, 0)


_tc_call = pl.pallas_call(
    _tc_body,
    out_shape=[
        jax.ShapeDtypeStruct((B, K1), jnp.float32),
        jax.ShapeDtypeStruct((B, K1), jnp.float32),
        jax.ShapeDtypeStruct((N, FEAT), jnp.float32),
        jax.ShapeDtypeStruct((N, FEAT), jnp.float32),
    ],
    in_specs=[
        pl.BlockSpec(memory_space=pltpu.MemorySpace.VMEM),
        pl.BlockSpec(memory_space=pltpu.MemorySpace.VMEM),
        pl.BlockSpec(memory_space=pltpu.MemorySpace.VMEM),
        pl.BlockSpec(memory_space=pltpu.MemorySpace.VMEM),
        pl.BlockSpec(memory_space=pltpu.MemorySpace.SMEM),
        pl.BlockSpec(memory_space=pltpu.MemorySpace.HBM),
        pl.BlockSpec(memory_space=pltpu.MemorySpace.HBM),
    ],
    out_specs=[
        pl.BlockSpec(memory_space=pltpu.MemorySpace.VMEM),
        pl.BlockSpec(memory_space=pltpu.MemorySpace.VMEM),
        pl.BlockSpec(memory_space=pltpu.MemorySpace.HBM),
        pl.BlockSpec(memory_space=pltpu.MemorySpace.HBM),
    ],
    scratch_shapes=[pltpu.SemaphoreType.DMA, pltpu.SemaphoreType.DMA],
    input_output_aliases={5: 2, 6: 3},
)


def kernel(l, ab, y, idx, memory_l, memory_ab):
    y = y.astype(jnp.int32)
    idx = idx.astype(jnp.int32)
    idx_p = jnp.concatenate(
        [idx, jnp.zeros((B, KP - K1), jnp.int32)], axis=1)
    sA, sB, updl, updab, ysc = _sc_call(
        l, ab, y.reshape(B // 16, 16), idx_p, memory_l, memory_ab)
    out_l, out_ab, new_l, new_ab = _tc_call(
        sA, sB, updl, updab, ysc, memory_l, memory_ab)
    return (out_l[..., None], out_ab[..., None], new_l, new_ab)
